# baseline (speedup 1.0000x reference)
import functools

import jax
import jax.numpy as jnp
from jax import lax
from jax.experimental import pallas as pl
from jax.experimental.pallas import tpu as pltpu
from jax.experimental.pallas import tpu_sc as plsc

_NUM_CORES = 2
_NUM_SUBCORES = 16
_NUM_WORKERS = _NUM_CORES * _NUM_SUBCORES

_TBLK = 8


def _sc_gather(table, idx_pad, rows_per_worker):
    n_pad = idx_pad.shape[0]
    d = table.shape[1]
    mesh = plsc.VectorSubcoreMesh(core_axis_name="c", subcore_axis_name="s")

    @functools.partial(
        pl.kernel,
        mesh=mesh,
        out_type=jax.ShapeDtypeStruct((n_pad, d), jnp.float32),
        scratch_types=[
            pltpu.VMEM((rows_per_worker,), jnp.int32),
            pltpu.VMEM((rows_per_worker, d), jnp.float32),
            pltpu.SemaphoreType.DMA,
        ],
    )
    def gather_kernel(table_hbm, idx_hbm, out_hbm, idx_v, rows_v, sem):
        wid = lax.axis_index("s") * _NUM_CORES + lax.axis_index("c")
        base = wid * rows_per_worker
        pltpu.sync_copy(idx_hbm.at[pl.ds(base, rows_per_worker)], idx_v)
        pltpu.async_copy(table_hbm.at[idx_v], rows_v, sem).wait()
        pltpu.sync_copy(rows_v, out_hbm.at[pl.ds(base, rows_per_worker)])

    return gather_kernel(table, idx_pad)


def _add_body(x_ref, pe_ref, o_ref):
    o_ref[...] = x_ref[...] + pe_ref[...][:, None, :]


def kernel(x, frame_indices, frame_embed):
    b, t, p, d = x.shape
    bt = b * t

    align = 8 * _NUM_WORKERS
    bt_pad = ((bt + align - 1) // align) * align
    idx = frame_indices.reshape(bt).astype(jnp.int32)
    idx_pad = jnp.pad(idx, (0, bt_pad - bt))

    pe = _sc_gather(frame_embed, idx_pad, bt_pad // _NUM_WORKERS)
    pe = pe[:bt]

    x3 = x.reshape(bt, p, d)
    out = pl.pallas_call(
        _add_body,
        grid=(bt // _TBLK,),
        in_specs=[
            pl.BlockSpec((_TBLK, p, d), lambda i: (i, 0, 0)),
            pl.BlockSpec((_TBLK, d), lambda i: (i, 0)),
        ],
        out_specs=pl.BlockSpec((_TBLK, p, d), lambda i: (i, 0, 0)),
        out_shape=jax.ShapeDtypeStruct((bt, p, d), jnp.float32),
    )(x3, pe)
    return out.reshape(b, t, p, d)


# speedup vs baseline: 1.1315x; 1.1315x over previous
import functools

import jax
import jax.numpy as jnp
from jax import lax
from jax.experimental import pallas as pl
from jax.experimental.pallas import tpu as pltpu
from jax.experimental.pallas import tpu_sc as plsc

_NUM_CORES = 2
_NUM_SUBCORES = 16
_NUM_WORKERS = _NUM_CORES * _NUM_SUBCORES

_TBLK = 8


def _sc_gather(table, idx_pad, rows_per_worker):
    n_pad = idx_pad.shape[0]
    d = table.shape[1]
    mesh = plsc.VectorSubcoreMesh(core_axis_name="c", subcore_axis_name="s")

    @functools.partial(
        pl.kernel,
        mesh=mesh,
        out_type=jax.ShapeDtypeStruct((n_pad, d), jnp.float32),
        scratch_types=[
            pltpu.VMEM((rows_per_worker,), jnp.int32),
            pltpu.VMEM((rows_per_worker, d), jnp.float32),
            pltpu.SemaphoreType.DMA,
        ],
    )
    def gather_kernel(table_hbm, idx_hbm, out_hbm, idx_v, rows_v, sem):
        wid = lax.axis_index("s") * _NUM_CORES + lax.axis_index("c")
        base = wid * rows_per_worker
        pltpu.sync_copy(idx_hbm.at[pl.ds(base, rows_per_worker)], idx_v)
        pltpu.async_copy(table_hbm.at[idx_v], rows_v, sem).wait()
        pltpu.sync_copy(rows_v, out_hbm.at[pl.ds(base, rows_per_worker)])

    return gather_kernel(table, idx_pad)


def _add_body(x_ref, pe_ref, o_ref):
    o_ref[...] = x_ref[...] + pe_ref[...][None, :, None, :]


def kernel(x, frame_indices, frame_embed):
    b, t, p, d = x.shape
    bt = b * t

    align = 8 * _NUM_WORKERS
    bt_pad = ((bt + align - 1) // align) * align
    idx = frame_indices.reshape(bt).astype(jnp.int32)
    idx_pad = jnp.pad(idx, (0, bt_pad - bt))

    pe = _sc_gather(frame_embed, idx_pad, bt_pad // _NUM_WORKERS)

    t_blocks = t // _TBLK
    out = pl.pallas_call(
        _add_body,
        grid=(b, t_blocks),
        in_specs=[
            pl.BlockSpec((1, _TBLK, p, d), lambda ib, it: (ib, it, 0, 0)),
            pl.BlockSpec((_TBLK, d), lambda ib, it: (ib * t_blocks + it, 0)),
        ],
        out_specs=pl.BlockSpec((1, _TBLK, p, d), lambda ib, it: (ib, it, 0, 0)),
        out_shape=jax.ShapeDtypeStruct((b, t, p, d), jnp.float32),
    )(x, pe)
    return out


# speedup vs baseline: 1.3869x; 1.2257x over previous
import functools

import jax
import jax.numpy as jnp
from jax import lax
from jax.experimental import pallas as pl
from jax.experimental.pallas import tpu as pltpu
from jax.experimental.pallas import tpu_sc as plsc

_NUM_CORES = 2
_NUM_SUBCORES = 16
_NUM_WORKERS = _NUM_CORES * _NUM_SUBCORES

_TBLK = 40


def _sc_gather(table, idx_pad, rows_per_worker):
    n_pad = idx_pad.shape[0]
    d = table.shape[1]
    mesh = plsc.VectorSubcoreMesh(core_axis_name="c", subcore_axis_name="s")

    @functools.partial(
        pl.kernel,
        mesh=mesh,
        out_type=jax.ShapeDtypeStruct((n_pad, d), jnp.float32),
        scratch_types=[
            pltpu.VMEM((rows_per_worker,), jnp.int32),
            pltpu.VMEM((rows_per_worker, d), jnp.float32),
            pltpu.SemaphoreType.DMA,
        ],
    )
    def gather_kernel(table_hbm, idx_hbm, out_hbm, idx_v, rows_v, sem):
        wid = lax.axis_index("s") * _NUM_CORES + lax.axis_index("c")
        base = wid * rows_per_worker
        pltpu.sync_copy(idx_hbm.at[pl.ds(base, rows_per_worker)], idx_v)
        pltpu.async_copy(table_hbm.at[idx_v], rows_v, sem).wait()
        pltpu.sync_copy(rows_v, out_hbm.at[pl.ds(base, rows_per_worker)])

    return gather_kernel(table, idx_pad)


def _add_body(x_ref, pe_ref, o_ref):
    o_ref[...] = x_ref[...] + pe_ref[...][None, :, None, :]


def kernel(x, frame_indices, frame_embed):
    b, t, p, d = x.shape
    bt = b * t

    align = 8 * _NUM_WORKERS
    bt_pad = ((bt + align - 1) // align) * align
    idx = frame_indices.reshape(bt).astype(jnp.int32)
    idx_pad = jnp.pad(idx, (0, bt_pad - bt))

    pe = _sc_gather(frame_embed, idx_pad, bt_pad // _NUM_WORKERS)

    t_blocks = t // _TBLK
    out = pl.pallas_call(
        _add_body,
        grid=(b, t_blocks),
        in_specs=[
            pl.BlockSpec((1, _TBLK, p, d), lambda ib, it: (ib, it, 0, 0)),
            pl.BlockSpec((_TBLK, d), lambda ib, it: (ib * t_blocks + it, 0)),
        ],
        out_specs=pl.BlockSpec((1, _TBLK, p, d), lambda ib, it: (ib, it, 0, 0)),
        out_shape=jax.ShapeDtypeStruct((b, t, p, d), jnp.float32),
    )(x, pe)
    return out


# speedup vs baseline: 1.3880x; 1.0008x over previous
import functools

import jax
import jax.numpy as jnp
from jax import lax
from jax.experimental import pallas as pl
from jax.experimental.pallas import tpu as pltpu
from jax.experimental.pallas import tpu_sc as plsc

_NUM_CORES = 2
_NUM_SUBCORES = 16
_NUM_WORKERS = _NUM_CORES * _NUM_SUBCORES

_TBLK = 40


def _sc_gather(table, idx_pad, rows_per_worker):
    n_pad = idx_pad.shape[0]
    d = table.shape[1]
    mesh = plsc.VectorSubcoreMesh(core_axis_name="c", subcore_axis_name="s")

    @functools.partial(
        pl.kernel,
        mesh=mesh,
        out_type=jax.ShapeDtypeStruct((n_pad, d), jnp.float32),
        scratch_types=[
            pltpu.VMEM((rows_per_worker,), jnp.int32),
            pltpu.VMEM((rows_per_worker, d), jnp.float32),
            pltpu.SemaphoreType.DMA,
        ],
    )
    def gather_kernel(table_hbm, idx_hbm, out_hbm, idx_v, rows_v, sem):
        wid = lax.axis_index("s") * _NUM_CORES + lax.axis_index("c")
        base = wid * rows_per_worker
        pltpu.sync_copy(idx_hbm.at[pl.ds(base, rows_per_worker)], idx_v)
        pltpu.async_copy(table_hbm.at[idx_v], rows_v, sem).wait()
        pltpu.sync_copy(rows_v, out_hbm.at[pl.ds(base, rows_per_worker)])

    return gather_kernel(table, idx_pad)


def _add_body(x_ref, pe_ref, o_ref):
    o_ref[...] = x_ref[...] + pe_ref[...][None, :, None, :]


def kernel(x, frame_indices, frame_embed):
    b, t, p, d = x.shape
    bt = b * t

    align = 8 * _NUM_WORKERS
    bt_pad = ((bt + align - 1) // align) * align
    idx = frame_indices.reshape(bt).astype(jnp.int32)
    idx_pad = jnp.pad(idx, (0, bt_pad - bt))

    pe = _sc_gather(frame_embed, idx_pad, bt_pad // _NUM_WORKERS)

    t_blocks = t // _TBLK
    out = pl.pallas_call(
        _add_body,
        grid=(b, t_blocks),
        in_specs=[
            pl.BlockSpec((1, _TBLK, p, d), lambda ib, it: (ib, it, 0, 0)),
            pl.BlockSpec((_TBLK, d), lambda ib, it: (ib * t_blocks + it, 0)),
        ],
        out_specs=pl.BlockSpec((1, _TBLK, p, d), lambda ib, it: (ib, it, 0, 0)),
        out_shape=jax.ShapeDtypeStruct((b, t, p, d), jnp.float32),
        compiler_params=pltpu.CompilerParams(
            dimension_semantics=("parallel", "parallel"),
        ),
    )(x, pe)
    return out


# speedup vs baseline: 1.3989x; 1.0079x over previous
import functools

import jax
import jax.numpy as jnp
from jax import lax
from jax.experimental import pallas as pl
from jax.experimental.pallas import tpu as pltpu
from jax.experimental.pallas import tpu_sc as plsc

_NUM_CORES = 2
_NUM_SUBCORES = 16
_NUM_WORKERS = _NUM_CORES * _NUM_SUBCORES

_TBLK = 40
_NBUF = 3


def _sc_gather(table, idx_pad, rows_per_worker):
    n_pad = idx_pad.shape[0]
    d = table.shape[1]
    mesh = plsc.VectorSubcoreMesh(core_axis_name="c", subcore_axis_name="s")

    @functools.partial(
        pl.kernel,
        mesh=mesh,
        out_type=jax.ShapeDtypeStruct((n_pad, d), jnp.float32),
        scratch_types=[
            pltpu.VMEM((rows_per_worker,), jnp.int32),
            pltpu.VMEM((rows_per_worker, d), jnp.float32),
            pltpu.SemaphoreType.DMA,
        ],
    )
    def gather_kernel(table_hbm, idx_hbm, out_hbm, idx_v, rows_v, sem):
        wid = lax.axis_index("s") * _NUM_CORES + lax.axis_index("c")
        base = wid * rows_per_worker
        pltpu.sync_copy(idx_hbm.at[pl.ds(base, rows_per_worker)], idx_v)
        pltpu.async_copy(table_hbm.at[idx_v], rows_v, sem).wait()
        pltpu.sync_copy(rows_v, out_hbm.at[pl.ds(base, rows_per_worker)])

    return gather_kernel(table, idx_pad)


def _make_add_body(b, t, p, d, bt_pad):
    t_chunks = t // _TBLK
    nsteps = b * t_chunks

    def add_body(pe_ref, x_hbm, o_hbm, ibuf, obuf, isem, osem):
        s = pl.program_id(0)

        def in_copy(step, slot):
            ib = step // t_chunks
            t0 = (step % t_chunks) * _TBLK
            return pltpu.make_async_copy(
                x_hbm.at[ib, pl.ds(t0, _TBLK)], ibuf.at[slot], isem.at[slot])

        def out_copy(step, slot):
            ib = step // t_chunks
            t0 = (step % t_chunks) * _TBLK
            return pltpu.make_async_copy(
                obuf.at[slot], o_hbm.at[ib, pl.ds(t0, _TBLK)], osem.at[slot])

        @pl.when(s == 0)
        def _prologue():
            for k in range(_NBUF):
                in_copy(k, k).start()

        slot = jax.lax.rem(s, _NBUF)
        in_copy(s, slot).wait()

        @pl.when(s >= _NBUF)
        def _():
            out_copy(s - _NBUF, slot).wait()

        pe_rows = pe_ref[pl.ds(s * _TBLK, _TBLK), :]
        obuf[slot] = ibuf[slot] + pe_rows[:, None, :]
        out_copy(s, slot).start()

        @pl.when(s + _NBUF < nsteps)
        def _():
            in_copy(s + _NBUF, slot).start()

        @pl.when(s == nsteps - 1)
        def _epilogue():
            for k in range(_NBUF):
                step = nsteps - _NBUF + k
                out_copy(step, step % _NBUF).wait()

    return add_body, nsteps


def kernel(x, frame_indices, frame_embed):
    b, t, p, d = x.shape
    bt = b * t

    align = 8 * _NUM_WORKERS
    bt_pad = ((bt + align - 1) // align) * align
    idx = frame_indices.reshape(bt).astype(jnp.int32)
    idx_pad = jnp.pad(idx, (0, bt_pad - bt))

    pe = _sc_gather(frame_embed, idx_pad, bt_pad // _NUM_WORKERS)

    add_body, nsteps = _make_add_body(b, t, p, d, bt_pad)
    out = pl.pallas_call(
        add_body,
        grid=(nsteps,),
        in_specs=[
            pl.BlockSpec((bt_pad, d), lambda s: (0, 0)),
            pl.BlockSpec(memory_space=pl.ANY),
        ],
        out_specs=pl.BlockSpec(memory_space=pl.ANY),
        out_shape=jax.ShapeDtypeStruct((b, t, p, d), jnp.float32),
        scratch_shapes=[
            pltpu.VMEM((_NBUF, _TBLK, p, d), jnp.float32),
            pltpu.VMEM((_NBUF, _TBLK, p, d), jnp.float32),
            pltpu.SemaphoreType.DMA((_NBUF,)),
            pltpu.SemaphoreType.DMA((_NBUF,)),
        ],
        compiler_params=pltpu.CompilerParams(
            dimension_semantics=("arbitrary",),
            vmem_limit_bytes=100 * 1024 * 1024,
        ),
    )(pe, x)
    return out
